# Initial kernel scaffold; baseline (speedup 1.0000x reference)
import jax
import jax.numpy as jnp
from jax.experimental import pallas as pl


def kernel(input):
    raise NotImplementedError("write your pallas kernel here")


# speedup vs baseline: 11.7607x; 11.7607x over previous
import functools
import numpy as np
import jax
import jax.numpy as jnp
from jax import lax
from jax.experimental import pallas as pl
from jax.experimental.pallas import tpu as pltpu
from jax.experimental.pallas import tpu_sc as plsc

N = 300000
CF = 5
NC = 2
NS = 16
NW = NC * NS
P = 9376
NV = P // 16
PADN = NW * P
CHK = (P + 127) // 128
PADN2 = PADN + 128
NB = 8192
BLK = NB // NW
SENT = 1024 * 1024 * 40
MAXV = 30000
MAXP = 32
VOXR = MAXV * MAXP
VOXA = VOXR + 32
NUMA = MAXV + 32
WU = 960
UNIA = NW * WU

F32 = jnp.float32
I32 = jnp.int32

_mesh = lambda: plsc.VectorSubcoreMesh(core_axis_name="c", subcore_axis_name="s")
_CP = pltpu.CompilerParams(needs_layout_passes=False)


def _wid():
    return lax.axis_index("c") * NS + lax.axis_index("s")


def _iota():
    return lax.broadcasted_iota(I32, (16,), 0)


def _take(v, i):
    return jnp.take_along_axis(v, i, axis=0)


def _run_rank(d, iota):
    sk, sv = plsc.sort_key_val(d, iota)
    prev = _take(sk, jnp.maximum(iota - 1, 0))
    is_start = (iota == 0) | (sk != prev)
    nxt = _take(sk, jnp.minimum(iota + 1, 15))
    is_end = (iota == 15) | (sk != nxt)
    rs = plsc.cummax(jnp.where(is_start, iota, 0))
    return sk, sv, iota - rs, is_end


def _hist_bump(hist_ref, d, iota):
    sk, _, rank, is_end = _run_rank(d, iota)
    plsc.addupdate_scatter(hist_ref, [sk], rank + 1, mask=is_end)


def _positions(myoffs_ref, d, iota, tmp16_ref):
    sk, sv, rank, is_end = _run_rank(d, iota)
    base = plsc.load_gather(myoffs_ref, [sk])
    plsc.addupdate_scatter(myoffs_ref, [sk], rank + 1, mask=is_end)
    plsc.store_scatter(tmp16_ref, [sv], base + rank)
    return tmp16_ref[...]


@functools.partial(
    pl.kernel, mesh=_mesh(), compiler_params=_CP,
    out_type=(jax.ShapeDtypeStruct((PADN,), I32),
              jax.ShapeDtypeStruct((NW * NB,), I32)),
    scratch_types=[pltpu.VMEM((P,), F32), pltpu.VMEM((P,), F32),
                   pltpu.VMEM((P,), F32), pltpu.VMEM((P,), I32),
                   pltpu.VMEM((NB,), I32)],
)
def _stage_ids_hist(xs, ys, zs, ids, h1, xb, yb, zb, ob, hist):
    w = _wid()
    start = w * P
    iota = _iota()
    pltpu.sync_copy(xs.at[pl.ds(start, P)], xb)
    pltpu.sync_copy(ys.at[pl.ds(start, P)], yb)
    pltpu.sync_copy(zs.at[pl.ds(start, P)], zb)

    def zero_body(t, _):
        hist[pl.ds(16 * t, 16)] = jnp.zeros((16,), I32)
        return 0
    lax.fori_loop(0, NB // 16, zero_body, 0)

    rx = np.float32(-51.2)
    rz = np.float32(-5.0)
    vx = np.float32(0.1)
    vz = np.float32(0.2)

    def body(j, _):
        s = pl.ds(16 * j, 16)
        cfx = (xb[s] - rx) / vx
        cfy = (yb[s] - rx) / vx
        cfz = (zb[s] - rz) / vz
        valid = ((cfx >= 0.0) & (cfx < 1024.0)
                 & (cfy >= 0.0) & (cfy < 1024.0)
                 & (cfz >= 0.0) & (cfz < 40.0))
        lin = (cfz.astype(I32) * (1024 * 1024) + cfy.astype(I32) * 1024
               + cfx.astype(I32))
        lin = jnp.where(valid, lin, SENT)
        ob[s] = lin
        _hist_bump(hist, lin & (NB - 1), iota)
        return 0
    lax.fori_loop(0, NV, body, 0)
    pltpu.sync_copy(ob, ids.at[pl.ds(start, P)])
    pltpu.sync_copy(hist, h1.at[pl.ds(w * NB, NB)])


@functools.partial(
    pl.kernel, mesh=_mesh(), compiler_params=_CP,
    out_type=jax.ShapeDtypeStruct((NW * NB,), I32),
    scratch_types=[pltpu.VMEM((NB,), I32), pltpu.VMEM((NB,), I32),
                   pltpu.VMEM((NW, BLK), I32), pltpu.VMEM((NW, BLK), I32)],
)
def _stage_scan(h, offs, rowbuf, acc, blk, outblk):
    w = _wid()
    iota = _iota()

    def zero_body(t, _):
        acc[pl.ds(16 * t, 16)] = jnp.zeros((16,), I32)
        return 0
    lax.fori_loop(0, NB // 16, zero_body, 0)

    def addrow(r, _):
        pltpu.sync_copy(h.at[pl.ds(r * NB, NB)], rowbuf)

        def inner(t, _):
            s = pl.ds(16 * t, 16)
            acc[s] = acc[s] + rowbuf[s]
            return 0
        lax.fori_loop(0, NB // 16, inner, 0)
        return 0
    lax.fori_loop(0, NW, addrow, 0)

    def blocksum(b, carry):
        sv0, sv1 = carry

        def inner(t, s):
            return s + jnp.sum(acc[pl.ds(b * BLK + 16 * t, 16)])
        s = lax.fori_loop(0, BLK // 16, inner, jnp.int32(0))
        sv0 = jnp.where(iota == b, s, sv0)
        sv1 = jnp.where(iota + 16 == b, s, sv1)
        return sv0, sv1
    v0, v1 = lax.fori_loop(0, NW, blocksum,
                           (jnp.zeros((16,), I32), jnp.zeros((16,), I32)))
    base = (jnp.sum(jnp.where(iota < w, v0, 0))
            + jnp.sum(jnp.where(iota + 16 < w, v1, 0)))

    def stage_blk(r, _):
        pltpu.sync_copy(h.at[pl.ds(r * NB + w * BLK, BLK)], blk.at[r])
        return 0
    lax.fori_loop(0, NW, stage_blk, 0)

    def scan_body(dd, carry):
        col = jnp.full((16,), dd, I32)
        c0 = plsc.load_gather(blk, [iota, col])
        c1 = plsc.load_gather(blk, [iota + 16, col])
        s0 = jnp.sum(c0)
        e0 = jnp.cumsum(c0) - c0 + carry
        e1 = jnp.cumsum(c1) - c1 + (carry + s0)
        plsc.store_scatter(outblk, [iota, col], e0)
        plsc.store_scatter(outblk, [iota + 16, col], e1)
        return carry + s0 + jnp.sum(c1)
    lax.fori_loop(0, BLK, scan_body, base)

    def put(r, _):
        pltpu.sync_copy(outblk.at[r], offs.at[pl.ds(r * NB + w * BLK, BLK)])
        return 0
    lax.fori_loop(0, NW, put, 0)


def _make_permute(shift, with_idx_in):

    def body(*args):
        if with_idx_in:
            (ids_in, idx_in, offs, ids_out, idx_out,
             idsbuf, idxvals, myoffs, posbuf, tmp16, sem) = args
        else:
            (ids_in, offs, ids_out, idx_out,
             idsbuf, idxvals, myoffs, posbuf, tmp16, sem) = args
            idx_in = None
        w = _wid()
        start = w * P
        iota = _iota()
        pltpu.sync_copy(ids_in.at[pl.ds(start, P)], idsbuf.at[pl.ds(0, P)])
        if with_idx_in:
            pltpu.sync_copy(idx_in.at[pl.ds(start, P)], idxvals.at[pl.ds(0, P)])
        pltpu.sync_copy(offs.at[pl.ds(w * NB, NB)], myoffs)
        for jj in range(2, 8):
            posbuf[CHK - 1, pl.ds(16 * jj, 16)] = PADN + 16 * jj + iota

        def body_k(k, _):
            for jj in range(8):
                j16 = 128 * k + 16 * jj

                @pl.when(j16 < P)
                def _():
                    s = pl.ds(j16, 16)
                    v = idsbuf[s]
                    d = (v >> shift) & (NB - 1) if shift else v & (NB - 1)
                    poss = _positions(myoffs, d, iota, tmp16)
                    posbuf[k, pl.ds(16 * jj, 16)] = poss
                    if not with_idx_in:
                        idxvals[s] = start + j16 + iota
            c1 = pltpu.async_copy(idsbuf.at[pl.ds(128 * k, 128)],
                                  ids_out.at[posbuf.at[k]], sem)
            c2 = pltpu.async_copy(idxvals.at[pl.ds(128 * k, 128)],
                                  idx_out.at[posbuf.at[k]], sem)
            c1.wait()
            c2.wait()
            return 0
        lax.fori_loop(0, CHK, body_k, 0)

    return pl.kernel(
        body, mesh=_mesh(), compiler_params=_CP,
        out_type=(jax.ShapeDtypeStruct((PADN2,), I32),
                  jax.ShapeDtypeStruct((PADN2,), I32)),
        scratch_types=[pltpu.VMEM((CHK * 128,), I32),
                       pltpu.VMEM((CHK * 128,), I32),
                       pltpu.VMEM((NB,), I32),
                       pltpu.VMEM((CHK, 128), I32),
                       pltpu.VMEM((16,), I32),
                       pltpu.SemaphoreType.DMA],
    )


_stage_permute1 = _make_permute(0, False)
_stage_permute2 = _make_permute(13, True)


@functools.partial(
    pl.kernel, mesh=_mesh(), compiler_params=_CP,
    out_type=jax.ShapeDtypeStruct((NW * NB,), I32),
    scratch_types=[pltpu.VMEM((P,), I32), pltpu.VMEM((NB,), I32)],
)
def _stage_hist2(ids2, h2, cb, hist):
    w = _wid()
    start = w * P
    iota = _iota()
    pltpu.sync_copy(ids2.at[pl.ds(start, P)], cb)

    def zero_body(t, _):
        hist[pl.ds(16 * t, 16)] = jnp.zeros((16,), I32)
        return 0
    lax.fori_loop(0, NB // 16, zero_body, 0)

    def body(j, _):
        v = cb[pl.ds(16 * j, 16)]
        _hist_bump(hist, (v >> 13) & (NB - 1), iota)
        return 0
    lax.fori_loop(0, NV, body, 0)
    pltpu.sync_copy(hist, h2.at[pl.ds(w * NB, NB)])


@functools.partial(
    pl.kernel, mesh=_mesh(), compiler_params=_CP,
    out_type=jax.ShapeDtypeStruct((2 * NW * 32,), I32),
    scratch_types=[pltpu.VMEM((P,), I32), pltpu.VMEM((32,), I32),
                   pltpu.VMEM((32,), I32)],
)
def _stage_stats(ids3, stats, cb, pbuf, rowbuf):
    w = _wid()
    start = w * P
    iota = _iota()
    pbuf[pl.ds(16, 16)] = jnp.full((16,), -1, I32)

    @pl.when(w > 0)
    def _():
        off = pl.multiple_of(jnp.maximum(start - 32, 0), 32)
        pltpu.sync_copy(ids3.at[pl.ds(off, 32)], pbuf)

    pltpu.sync_copy(ids3.at[pl.ds(start, P)], cb)
    prevc0 = pbuf[pl.ds(16, 16)][15]

    def body(j, carry):
        nstart, last, prevc = carry
        v = cb[pl.ds(16 * j, 16)]
        pg = plsc.load_gather(cb, [jnp.maximum(16 * j - 1 + iota, 0)])
        prevs = jnp.where(iota == 0, prevc, pg)
        st = v != prevs
        ns = v != SENT
        nstart = nstart + jnp.sum((st & ns).astype(I32))
        pos = start + 16 * j + iota
        last = jnp.maximum(last, jnp.max(jnp.where(st, pos, -1)))
        prevc = jnp.sum(jnp.where(iota == 15, v, 0))
        return nstart, last, prevc
    nstart, last, _ = lax.fori_loop(
        0, NV, body, (jnp.int32(0), jnp.int32(-1), prevc0))
    rowbuf[pl.ds(0, 16)] = jnp.full((16,), nstart, I32)
    rowbuf[pl.ds(16, 16)] = jnp.full((16,), nstart, I32)
    pltpu.sync_copy(rowbuf, stats.at[pl.ds(w * 32, 32)])
    rowbuf[pl.ds(0, 16)] = jnp.full((16,), last, I32)
    rowbuf[pl.ds(16, 16)] = jnp.full((16,), last, I32)
    pltpu.sync_copy(rowbuf, stats.at[pl.ds((NW + w) * 32, 32)])


@functools.partial(
    pl.kernel, mesh=_mesh(), compiler_params=_CP,
    out_type=(),
    scratch_types=[pltpu.VMEM((P + 32,), I32), pltpu.VMEM((P,), I32),
                   pltpu.VMEM((32,), I32), pltpu.VMEM((2 * NW * 32,), I32),
                   pltpu.VMEM((CHK * 128,), I32), pltpu.VMEM((CHK * 128,), I32),
                   pltpu.VMEM((CF, 128), I32), pltpu.VMEM((CF, 128), I32),
                   pltpu.VMEM((CHK, 128), I32), pltpu.VMEM((CHK * 128,), I32),
                   pltpu.VMEM((CHK * 128,), I32), pltpu.VMEM((128 * CF,), F32),
                   pltpu.SemaphoreType.DMA],
)
def _stage_output(ids3, idx3, points, stats_in, vox_ref, num_ref, uniq_ref,
                  cbe, ib, pbuf, sbuf, sidx, srow, itmp, rtmp, svid, numv,
                  univ, prow, sem):
    w = _wid()
    start = w * P
    iota = _iota()
    pltpu.sync_copy(stats_in, sbuf)
    pbuf[pl.ds(16, 16)] = jnp.full((16,), -1, I32)

    @pl.when(w > 0)
    def _():
        off = pl.multiple_of(jnp.maximum(start - 32, 0), 32)
        pltpu.sync_copy(ids3.at[pl.ds(off, 32)], pbuf)

    @pl.when(w < NW - 1)
    def _():
        pltpu.sync_copy(ids3.at[pl.ds(start, P + 32)], cbe)

    @pl.when(w == NW - 1)
    def _():
        pltpu.sync_copy(ids3.at[pl.ds(start, P)], cbe.at[pl.ds(0, P)])
        cbe[pl.ds(P, 16)] = jnp.full((16,), SENT, I32)
        cbe[pl.ds(P + 16, 16)] = jnp.full((16,), SENT, I32)

    pltpu.sync_copy(idx3.at[pl.ds(start, P)], ib)

    def prefill(k, _):
        for jj in range(8):
            pat = (iota + 16 * jj) & 31
            s = pl.ds(128 * k + 16 * jj, 16)
            sidx[s] = jnp.zeros((16,), I32)
            srow[s] = VOXR + pat
            svid[k, pl.ds(16 * jj, 16)] = MAXV + pat
        return 0
    lax.fori_loop(0, CHK, prefill, 0)

    nv0 = plsc.load_gather(sbuf, [iota * 32])
    nv1 = plsc.load_gather(sbuf, [(iota + 16) * 32])
    lv0 = plsc.load_gather(sbuf, [(iota + 32) * 32])
    lv1 = plsc.load_gather(sbuf, [(iota + 48) * 32])
    base = (jnp.sum(jnp.where(iota < w, nv0, 0))
            + jnp.sum(jnp.where(iota + 16 < w, nv1, 0)))
    rsc0 = jnp.maximum(jnp.max(jnp.where(iota < w, lv0, -1)),
                       jnp.max(jnp.where(iota + 16 < w, lv1, -1)))
    prevc0 = pbuf[pl.ds(16, 16)][15]

    def body(j, carry):
        cnt, ncnt, vrun, rsc, prevc = carry
        s = pl.ds(16 * j, 16)
        v = cbe[s]
        nxt = plsc.load_gather(cbe, [16 * j + 1 + iota])
        pg = plsc.load_gather(cbe, [jnp.maximum(16 * j - 1 + iota, 0)])
        prevs = jnp.where(iota == 0, prevc, pg)
        st = v != prevs
        ns = v != SENT
        sn = (st & ns).astype(I32)
        vidx = vrun + jnp.cumsum(sn) - 1
        pos = start + 16 * j + iota
        rs = jnp.maximum(plsc.cummax(jnp.where(st, pos, -1)),
                         jnp.full((16,), rsc, I32))
        rank = pos - rs
        ok = ns & (rank < MAXP) & (vidx < MAXV)
        oki = ok.astype(I32)
        cpos = cnt + jnp.cumsum(oki) - 1
        plsc.store_scatter(sidx, [cpos], ib[s], mask=ok)
        plsc.store_scatter(srow, [cpos], vidx * MAXP + rank, mask=ok)
        islast = v != nxt
        okn = ns & (vidx < MAXV) & ((islast & (rank < MAXP))
                                    | (rank == MAXP - 1))
        okni = okn.astype(I32)
        npos = ncnt + jnp.cumsum(okni) - 1
        plsc.store_scatter(svid, [npos >> 7, npos & 127], vidx, mask=okn)
        plsc.store_scatter(numv, [npos], rank + 1, mask=okn)
        plsc.store_scatter(univ, [npos], v, mask=okn)
        return (cnt + jnp.sum(oki), ncnt + jnp.sum(okni),
                vrun + jnp.sum(sn), jnp.max(rs),
                jnp.sum(jnp.where(iota == 15, v, 0)))
    cnt, ncnt, _, _, _ = lax.fori_loop(
        0, NV, body, (jnp.int32(0), jnp.int32(0), base, rsc0, prevc0))

    def dma_vox(k, _):
        for jj in range(8):
            s = pl.ds(128 * k + 16 * jj, 16)
            i5 = sidx[s] * CF
            r5 = srow[s] * CF
            for c in range(CF):
                itmp[c, pl.ds(16 * jj, 16)] = i5 + c
                rtmp[c, pl.ds(16 * jj, 16)] = r5 + c
        gs = [pltpu.async_copy(points.at[itmp.at[c]],
                               prow.at[pl.ds(128 * c, 128)], sem)
              for c in range(CF)]
        for g in gs:
            g.wait()
        ss = [pltpu.async_copy(prow.at[pl.ds(128 * c, 128)],
                               vox_ref.at[rtmp.at[c]], sem)
              for c in range(CF)]
        for s_ in ss:
            s_.wait()
        return 0
    lax.fori_loop(0, (cnt + 127) >> 7, dma_vox, 0)

    def dma_num(k, _):
        a = pltpu.async_copy(numv.at[pl.ds(128 * k, 128)],
                             num_ref.at[svid.at[k]], sem)
        b = pltpu.async_copy(univ.at[pl.ds(128 * k, 128)],
                             uniq_ref.at[svid.at[k]], sem)
        a.wait()
        b.wait()
        return 0
    lax.fori_loop(0, (ncnt + 127) >> 7, dma_num, 0)


@functools.partial(
    pl.kernel, mesh=_mesh(), compiler_params=_CP,
    out_type=jax.ShapeDtypeStruct((UNIA * 4,), I32),
    scratch_types=[pltpu.VMEM((WU,), I32), pltpu.VMEM((WU * 4,), I32)],
)
def _stage_coords(uniq_ref, coors, ubuf, cbuf):
    w = _wid()
    start = w * WU
    iota = _iota()
    pltpu.sync_copy(uniq_ref.at[pl.ds(start, WU)], ubuf)

    def body(j, _):
        u = ubuf[pl.ds(16 * j, 16)]
        m = u == SENT
        li = (16 * j + iota) * 4
        plsc.store_scatter(cbuf, [li], jnp.where(m, -1, u >> 20))
        plsc.store_scatter(cbuf, [li + 1], jnp.where(m, -1, (u >> 10) & 1023))
        plsc.store_scatter(cbuf, [li + 2], jnp.where(m, -1, u & 1023))
        plsc.store_scatter(cbuf, [li + 3], jnp.zeros((16,), I32))
        return 0
    lax.fori_loop(0, WU // 16, body, 0)
    pltpu.sync_copy(cbuf, coors.at[pl.ds(start * 4, WU * 4)])


def kernel(input):
    pts = input.astype(F32)
    pad = (0, PADN - N)
    xs = jnp.pad(pts[:, 0], pad, constant_values=-1e4)
    ys = jnp.pad(pts[:, 1], pad, constant_values=-1e4)
    zs = jnp.pad(pts[:, 2], pad, constant_values=-1e4)
    points_flat = pts.reshape(-1)
    ids, h1 = _stage_ids_hist(xs, ys, zs)
    offs1 = _stage_scan(h1)
    ids2, idx2 = _stage_permute1(ids, offs1)
    h2 = _stage_hist2(ids2)
    offs2 = _stage_scan(h2)
    ids3, idx3 = _stage_permute2(ids2, idx2, offs2)
    stats = _stage_stats(ids3)
    vox_ref = jax.new_ref(jnp.zeros((VOXA * CF,), F32))
    num_ref = jax.new_ref(jnp.zeros((NUMA,), I32))
    uniq_ref = jax.new_ref(jnp.full((UNIA,), SENT, I32))
    _stage_output(ids3, idx3, points_flat, stats, vox_ref, num_ref, uniq_ref)
    coors4 = _stage_coords(uniq_ref)
    voxels = vox_ref[...][:VOXR * CF].reshape(MAXV, MAXP, CF)
    coors = coors4.reshape(UNIA, 4)[:MAXV, :3]
    nump = num_ref[...][:MAXV]
    return voxels, coors, nump
